# Initial kernel scaffold; baseline (speedup 1.0000x reference)
import jax
import jax.numpy as jnp
from jax.experimental import pallas as pl


def kernel(x, edge_index, edge_attr, t, batch, condition, params):
    raise NotImplementedError("write your pallas kernel here")


# speedup vs baseline: 2.0352x; 2.0352x over previous
import jax
import jax.numpy as jnp
from jax import lax
from jax.experimental import pallas as pl
from jax.experimental.pallas import tpu as pltpu
from jax.experimental.pallas import tpu_sc as plsc

N = 50000
NP = 50176
E = 800000
EP = 819200
H = 64
B = 16
NBLK = 3136
EBLK = 4096

_f32 = jnp.float32


def _mesh():
    return plsc.VectorSubcoreMesh(core_axis_name="c", subcore_axis_name="s")


_SC_PARAMS = pltpu.CompilerParams(use_tc_tiling_on_sc=False)


def _count_edges(dst2):
    @pl.kernel(
        out_type=jax.ShapeDtypeStruct((2, NP), _f32),
        mesh=_mesh(),
        scratch_types=[
            pltpu.VMEM((8, 128), jnp.int32),
            pltpu.VMEM((128,), _f32),
            pltpu.VMEM((3136,), _f32),
            pltpu.VMEM_SHARED((NP,), _f32),
        ],
        compiler_params=_SC_PARAMS,
    )
    def k(dst_hbm, cnt_hbm, idxb, ones, zb, acc):
        c = lax.axis_index("c")
        s = lax.axis_index("s")
        wid = s * 2 + c

        @pl.loop(0, 3136, step=16)
        def _(i):
            zb[pl.ds(i, 16)] = jnp.zeros((16,), _f32)

        @pl.loop(0, 128, step=16)
        def _(i):
            ones[pl.ds(i, 16)] = jnp.ones((16,), _f32)

        pltpu.sync_copy(zb, acc.at[pl.ds(s * 3136, 3136)])
        plsc.subcore_barrier()

        @pl.loop(0, 25)
        def _(kk):
            pltpu.sync_copy(dst_hbm.at[pl.ds(wid * 200 + kk * 8, 8)], idxb)
            for j in range(8):
                pltpu.sync_copy(ones, acc.at[idxb.at[j]], add=True)

        plsc.subcore_barrier()
        pltpu.sync_copy(
            acc.at[pl.ds(s * 3136, 3136)], cnt_hbm.at[c, pl.ds(s * 3136, 3136)]
        )

    return k(dst2)


def _gather_uv(u, v, dst2, src2):
    @pl.kernel(
        out_type=jax.ShapeDtypeStruct((EP, H), _f32),
        mesh=_mesh(),
        scratch_types=[
            pltpu.VMEM((4, 128), jnp.int32),
            pltpu.VMEM((4, 128), jnp.int32),
            pltpu.VMEM((512, H), _f32),
            pltpu.VMEM((512, H), _f32),
        ],
        compiler_params=_SC_PARAMS,
    )
    def k(u_hbm, v_hbm, dst_hbm, src_hbm, g_hbm, db, sb, ub, vb):
        c = lax.axis_index("c")
        s = lax.axis_index("s")
        wid = s * 2 + c
        row0 = wid * 200

        @pl.loop(0, 50)
        def _(kk):
            pltpu.sync_copy(dst_hbm.at[pl.ds(row0 + kk * 4, 4)], db)
            pltpu.sync_copy(src_hbm.at[pl.ds(row0 + kk * 4, 4)], sb)
            for j in range(4):
                pltpu.sync_copy(u_hbm.at[db.at[j]], ub.at[pl.ds(j * 128, 128)])
                pltpu.sync_copy(v_hbm.at[sb.at[j]], vb.at[pl.ds(j * 128, 128)])

            @pl.loop(0, 512)
            def _(r):
                for f in range(4):
                    sl = pl.ds(f * 16, 16)
                    ub[r, sl] = ub[r, sl] + vb[r, sl]

            pltpu.sync_copy(
                ub, g_hbm.at[pl.ds(wid * 25600 + kk * 512, 512)]
            )

    return k(u, v, dst2, src2)


def _scatter_add(r, dst2):
    @pl.kernel(
        out_type=jax.ShapeDtypeStruct((NP, H), _f32),
        mesh=_mesh(),
        scratch_types=[
            pltpu.VMEM((4, 128), jnp.int32),
            pltpu.VMEM((512, 32), _f32),
            pltpu.VMEM((112, 32), _f32),
            pltpu.VMEM_SHARED((NP, 32), _f32),
        ],
        compiler_params=_SC_PARAMS,
    )
    def k(r_hbm, dst_hbm, s_hbm, idxb, rb, zb, acc):
        c = lax.axis_index("c")
        s = lax.axis_index("s")

        @pl.loop(0, 112)
        def _(i):
            for f in range(2):
                zb[i, pl.ds(f * 16, 16)] = jnp.zeros((16,), _f32)

        @pl.loop(0, 28)
        def _(i):
            pltpu.sync_copy(zb, acc.at[pl.ds(s * 3136 + i * 112, 112)])

        plsc.subcore_barrier()

        @pl.loop(0, 100)
        def _(kk):
            e0 = s * 51200 + kk * 512
            pltpu.sync_copy(dst_hbm.at[pl.ds(s * 400 + kk * 4, 4)], idxb)
            pltpu.sync_copy(r_hbm.at[pl.ds(e0, 512), pl.ds(c * 32, 32)], rb)
            for j in range(4):
                pltpu.sync_copy(
                    rb.at[pl.ds(j * 128, 128)], acc.at[idxb.at[j]], add=True
                )

        plsc.subcore_barrier()
        pltpu.sync_copy(
            acc.at[pl.ds(s * 3136, 3136)],
            s_hbm.at[pl.ds(s * 3136, 3136), pl.ds(c * 32, 32)],
        )

    return k(r, dst2)


def _h0(xp, batchp, t2, condp, wi1, bi1, wi2, bi2, wt1, bt1, wt2, bt2,
        wc1, bc1, wc2, bc2):
    def body(x_ref, b_ref, t_ref, c_ref, wi1r, bi1r, wi2r, bi2r, wt1r, bt1r,
             wt2r, bt2r, wc1r, bc1r, wc2r, bc2r, o_ref):
        tz = jnp.maximum(t_ref[...] * wt1r[...] + bt1r[...], 0.0)
        tf = jnp.dot(tz, wt2r[...], preferred_element_type=_f32) + bt2r[...]
        cz = jnp.maximum(
            jnp.dot(c_ref[...], wc1r[...], preferred_element_type=_f32)
            + bc1r[...], 0.0)
        cf = jnp.dot(cz, wc2r[...], preferred_element_type=_f32) + bc2r[...]
        tfc = tf + cf
        z = jnp.maximum(
            jnp.dot(x_ref[...], wi1r[...], preferred_element_type=_f32)
            + bi1r[...], 0.0)
        h = jnp.dot(z, wi2r[...], preferred_element_type=_f32) + bi2r[...]
        oh = (b_ref[...] == lax.broadcasted_iota(jnp.int32, (1, B), 1)
              ).astype(_f32)
        o_ref[...] = h + jnp.dot(oh, tfc, preferred_element_type=_f32)

    full = lambda a: pl.BlockSpec(a.shape, lambda i: (0,) * a.ndim)
    return pl.pallas_call(
        body,
        grid=(NP // NBLK,),
        in_specs=[
            pl.BlockSpec((NBLK, 8), lambda i: (i, 0)),
            pl.BlockSpec((NBLK, 1), lambda i: (i, 0)),
            full(t2), full(condp),
            full(wi1), full(bi1), full(wi2), full(bi2),
            full(wt1), full(bt1), full(wt2), full(bt2),
            full(wc1), full(bc1), full(wc2), full(bc2),
        ],
        out_specs=pl.BlockSpec((NBLK, H), lambda i: (i, 0)),
        out_shape=jax.ShapeDtypeStruct((NP, H), _f32),
    )(xp, batchp, t2, condp, wi1, bi1, wi2, bi2, wt1, bt1, wt2, bt2,
      wc1, bc1, wc2, bc2)


def _edge_relu(g, eap, w1r):
    def body(g_ref, ea_ref, w_ref, o_ref):
        o_ref[...] = jnp.maximum(g_ref[...] + ea_ref[...] * w_ref[...], 0.0)

    return pl.pallas_call(
        body,
        grid=(EP // EBLK,),
        in_specs=[
            pl.BlockSpec((EBLK, H), lambda i: (i, 0)),
            pl.BlockSpec((EBLK, 1), lambda i: (i, 0)),
            pl.BlockSpec((1, H), lambda i: (0, 0)),
        ],
        out_specs=pl.BlockSpec((EBLK, H), lambda i: (i, 0)),
        out_shape=jax.ShapeDtypeStruct((EP, H), _f32),
    )(g, eap, w1r)


def _epi_uv(sagg, h, cp0, cp1, w2, b2, a1, b1, a2, want_uv):
    def body(*refs):
        if want_uv:
            (s_ref, h_ref, c0_ref, c1_ref, w2r, b2r, a1r, b1r, a2r,
             ho, uo, vo) = refs
        else:
            s_ref, h_ref, c0_ref, c1_ref, w2r, b2r, ho = refs
        cnt = c0_ref[...] + c1_ref[...]
        invc = 1.0 / jnp.maximum(cnt, 1.0)
        hasb = (cnt > 0.0).astype(_f32)
        q = (jnp.dot(s_ref[...] * invc, w2r[...], preferred_element_type=_f32)
             + b2r[...] * hasb)
        hn = jnp.maximum(q, 0.0) + h_ref[...]
        ho[...] = hn
        if want_uv:
            uo[...] = jnp.dot(hn, a1r[...], preferred_element_type=_f32) + b1r[...]
            vo[...] = jnp.dot(hn, a2r[...], preferred_element_type=_f32)

    nb = pl.BlockSpec((NBLK, H), lambda i: (i, 0))
    cb = pl.BlockSpec((NBLK, 1), lambda i: (i, 0))
    full = lambda a: pl.BlockSpec(a.shape, lambda i: (0,) * a.ndim)
    if want_uv:
        in_specs = [nb, nb, cb, cb, full(w2), full(b2), full(a1), full(b1),
                    full(a2)]
        args = (sagg, h, cp0, cp1, w2, b2, a1, b1, a2)
        out_specs = [nb, nb, nb]
        out_shape = [jax.ShapeDtypeStruct((NP, H), _f32)] * 3
    else:
        in_specs = [nb, nb, cb, cb, full(w2), full(b2)]
        args = (sagg, h, cp0, cp1, w2, b2)
        out_specs = nb
        out_shape = jax.ShapeDtypeStruct((NP, H), _f32)
    return pl.pallas_call(
        body, grid=(NP // NBLK,), in_specs=in_specs, out_specs=out_specs,
        out_shape=out_shape)(*args)


def _uv(h, a1, b1, a2):
    def body(h_ref, a1r, b1r, a2r, uo, vo):
        hn = h_ref[...]
        uo[...] = jnp.dot(hn, a1r[...], preferred_element_type=_f32) + b1r[...]
        vo[...] = jnp.dot(hn, a2r[...], preferred_element_type=_f32)

    nb = pl.BlockSpec((NBLK, H), lambda i: (i, 0))
    full = lambda a: pl.BlockSpec(a.shape, lambda i: (0,) * a.ndim)
    return pl.pallas_call(
        body,
        grid=(NP // NBLK,),
        in_specs=[nb, full(a1), full(b1), full(a2)],
        out_specs=[nb, nb],
        out_shape=[jax.ShapeDtypeStruct((NP, H), _f32)] * 2,
    )(h, a1, b1, a2)


def _out_mlp(h, w1, b1, w2p, b2p):
    def body(h_ref, w1r, b1r, w2r, b2r, o_ref):
        z = jnp.maximum(
            jnp.dot(h_ref[...], w1r[...], preferred_element_type=_f32)
            + b1r[...], 0.0)
        o_ref[...] = jnp.dot(z, w2r[...], preferred_element_type=_f32) + b2r[...]

    nb = pl.BlockSpec((NBLK, H), lambda i: (i, 0))
    full = lambda a: pl.BlockSpec(a.shape, lambda i: (0,) * a.ndim)
    return pl.pallas_call(
        body,
        grid=(NP // NBLK,),
        in_specs=[nb, full(w1), full(b1), full(w2p), full(b2p)],
        out_specs=pl.BlockSpec((NBLK, 8), lambda i: (i, 0)),
        out_shape=jax.ShapeDtypeStruct((NP, 8), _f32),
    )(h, w1, b1, w2p, b2p)


def kernel(x, edge_index, edge_attr, t, batch, condition, params):
    f32 = _f32
    row = lambda b: b.reshape(1, -1).astype(f32)

    src = edge_index[0]
    dst = edge_index[1]
    pad_dst = N + (jnp.arange(EP - E, dtype=jnp.int32) % (NP - N))
    dst2 = jnp.concatenate([dst, pad_dst]).reshape(-1, 128)
    src2 = jnp.concatenate([src, jnp.zeros((EP - E,), jnp.int32)]).reshape(-1, 128)
    eap = jnp.concatenate([edge_attr, jnp.zeros((EP - E, 1), f32)], axis=0)

    xp = jnp.pad(x, ((0, NP - N), (0, 8 - x.shape[1])))
    batchp = jnp.pad(batch, (0, NP - N)).reshape(-1, 1)
    t2 = t.reshape(-1, 1)
    condp = jnp.pad(condition, ((0, 0), (0, 4)))

    ip = params["input_mlp"]
    tp = params["time_mlp"]
    cp = params["cond_mlp"]
    op = params["output_mlp"]

    wi1 = jnp.pad(ip[0]["W"], ((0, 8 - ip[0]["W"].shape[0]), (0, 0)))
    wc1 = jnp.pad(cp[0]["W"], ((0, 4), (0, 0)))
    wo2 = jnp.pad(op[1]["W"], ((0, 0), (0, 8 - op[1]["W"].shape[1])))
    bo2 = jnp.pad(op[1]["b"], (0, 8 - op[1]["b"].shape[0])).reshape(1, -1)

    cnt = _count_edges(dst2)
    cp0 = cnt[0].reshape(-1, 1)
    cp1 = cnt[1].reshape(-1, 1)

    h = _h0(xp, batchp, t2, condp,
            wi1, row(ip[0]["b"]), ip[1]["W"], row(ip[1]["b"]),
            tp[0]["W"], row(tp[0]["b"]), tp[1]["W"], row(tp[1]["b"]),
            wc1, row(cp[0]["b"]), cp[1]["W"], row(cp[1]["b"]))

    convs = params["convs"]
    a1 = convs[0][0]["W"][:H]
    a2 = convs[0][0]["W"][H:2 * H]
    u, v = _uv(h, a1, row(convs[0][0]["b"]), a2)

    for l in range(6):
        w1r = convs[l][0]["W"][2 * H:2 * H + 1]
        g = _gather_uv(u, v, dst2, src2)
        r = _edge_relu(g, eap, w1r)
        sagg = _scatter_add(r, dst2)
        w2 = convs[l][1]["W"]
        b2 = row(convs[l][1]["b"])
        if l < 5:
            na1 = convs[l + 1][0]["W"][:H]
            na2 = convs[l + 1][0]["W"][H:2 * H]
            h, u, v = _epi_uv(sagg, h, cp0, cp1, w2, b2,
                              na1, row(convs[l + 1][0]["b"]), na2, True)
        else:
            h = _epi_uv(sagg, h, cp0, cp1, w2, b2, None, None, None, False)

    outp = _out_mlp(h, op[0]["W"], row(op[0]["b"]), wo2, bo2)
    return outp[:N, :3]


# speedup vs baseline: 2.2498x; 1.1055x over previous
import jax
import jax.numpy as jnp
from jax import lax
from jax.experimental import pallas as pl
from jax.experimental.pallas import tpu as pltpu
from jax.experimental.pallas import tpu_sc as plsc

N = 50000
NP = 50176
E = 800000
EP = 819200
H = 64
B = 16
NBLK = 3136
EBLK = 4096

_f32 = jnp.float32


def _mesh():
    return plsc.VectorSubcoreMesh(core_axis_name="c", subcore_axis_name="s")


_SC_PARAMS = pltpu.CompilerParams(use_tc_tiling_on_sc=False)


def _count_edges(dst2):
    @pl.kernel(
        out_type=jax.ShapeDtypeStruct((2, NP, 32), _f32),
        mesh=_mesh(),
        scratch_types=[
            pltpu.VMEM((2, 2, 128), jnp.int32),
            pltpu.VMEM((128, 32), _f32),
            pltpu.VMEM((112, 32), _f32),
            pltpu.VMEM_SHARED((NP, 32), _f32),
            pltpu.SemaphoreType.DMA((2,)),
            pltpu.SemaphoreType.DMA((2,)),
        ],
        compiler_params=_SC_PARAMS,
    )
    def k(dst_hbm, cnt_hbm, idxb, ones, zb, acc, semi, semsc):
        c = lax.axis_index("c")
        s = lax.axis_index("s")
        row0 = c * 3200 + s * 200

        @pl.loop(0, 112)
        def _(i):
            for f in range(2):
                zb[i, pl.ds(f * 16, 16)] = jnp.zeros((16,), _f32)

        @pl.loop(0, 128)
        def _(i):
            for f in range(2):
                ones[i, pl.ds(f * 16, 16)] = jnp.ones((16,), _f32)

        @pl.loop(0, 28)
        def _(i):
            pltpu.sync_copy(zb, acc.at[pl.ds(s * 3136 + i * 112, 112)])

        plsc.subcore_barrier()

        def in_cp(kk, b):
            return pltpu.make_async_copy(
                dst_hbm.at[pl.ds(row0 + kk * 2, 2)], idxb.at[b], semi.at[b])

        def start_sc(b):
            for j in range(2):
                pltpu.async_copy(ones, acc.at[idxb.at[b].at[j]],
                                 semsc.at[b], add=True)

        def wait_sc(b):
            for j in range(2):
                pltpu.make_async_copy(ones, acc.at[idxb.at[b].at[j]],
                                      semsc.at[b]).wait()

        def body(kk, b, first, prefetch):
            in_cp(0, b).wait()
            start_sc(b)
            if not first:
                wait_sc(1 - b)
            if prefetch:
                in_cp(kk + 1, 1 - b).start()

        in_cp(0, 0).start()
        body(0, 0, True, True)

        @pl.loop(0, 49)
        def _(i):
            body(1 + i * 2, 1, False, True)
            body(2 + i * 2, 0, False, True)

        body(99, 1, False, False)
        wait_sc(1)

        plsc.subcore_barrier()
        pltpu.sync_copy(
            acc.at[pl.ds(s * 3136, 3136)],
            cnt_hbm.at[c, pl.ds(s * 3136, 3136)],
        )

    return k(dst2)


def _gather_uv(u, v, dst2, src2):
    @pl.kernel(
        out_type=jax.ShapeDtypeStruct((EP, H), _f32),
        mesh=_mesh(),
        scratch_types=[
            pltpu.VMEM((2, 2, 128), jnp.int32),
            pltpu.VMEM((2, 2, 128), jnp.int32),
            pltpu.VMEM((2, 256, H), _f32),
            pltpu.VMEM((2, 256, H), _f32),
            pltpu.SemaphoreType.DMA((2,)),
            pltpu.SemaphoreType.DMA((2,)),
            pltpu.SemaphoreType.DMA((2,)),
        ],
        compiler_params=_SC_PARAMS,
    )
    def k(u_hbm, v_hbm, dst_hbm, src_hbm, g_hbm, db, sb, ub, vb,
          semi, semg, semo):
        c = lax.axis_index("c")
        s = lax.axis_index("s")
        wid = s * 2 + c
        row0 = wid * 200
        e0 = wid * 25600

        def idx_cps(kk, b):
            r = row0 + kk * 2
            return (
                pltpu.make_async_copy(dst_hbm.at[pl.ds(r, 2)], db.at[b],
                                      semi.at[b]),
                pltpu.make_async_copy(src_hbm.at[pl.ds(r, 2)], sb.at[b],
                                      semi.at[b]),
            )

        def g_cps(b):
            cps = []
            for j in range(2):
                cps.append(pltpu.make_async_copy(
                    u_hbm.at[db.at[b].at[j]],
                    ub.at[b].at[pl.ds(j * 128, 128)], semg.at[b]))
                cps.append(pltpu.make_async_copy(
                    v_hbm.at[sb.at[b].at[j]],
                    vb.at[b].at[pl.ds(j * 128, 128)], semg.at[b]))
            return cps

        def out_cp(kk, b):
            return pltpu.make_async_copy(
                ub.at[b], g_hbm.at[pl.ds(e0 + kk * 256, 256)], semo.at[b])

        def body(kk, b, first, prefetch):
            for cp in idx_cps(0, b):
                cp.wait()
            if not first:
                out_cp(0, b).wait()
            for cp in g_cps(b):
                cp.start()
            if prefetch:
                for cp in idx_cps(kk + 1, 1 - b):
                    cp.start()
            for cp in g_cps(b):
                cp.wait()

            @pl.loop(0, 256, step=4)
            def _(r):
                for rr in range(4):
                    for f in range(4):
                        sl = pl.ds(f * 16, 16)
                        ub[b, r + rr, sl] = ub[b, r + rr, sl] + vb[b, r + rr, sl]

            out_cp(kk, b).start()

        for cp in idx_cps(0, 0):
            cp.start()
        body(0, 0, True, True)
        body(1, 1, True, True)

        @pl.loop(0, 48)
        def _(i):
            body(2 + i * 2, 0, False, True)
            body(3 + i * 2, 1, False, True)

        body(98, 0, False, True)
        body(99, 1, False, False)
        out_cp(0, 0).wait()
        out_cp(0, 1).wait()

    return k(u, v, dst2, src2)


def _scatter_add(r, dst2):
    @pl.kernel(
        out_type=jax.ShapeDtypeStruct((NP, H), _f32),
        mesh=_mesh(),
        scratch_types=[
            pltpu.VMEM((2, 2, 128), jnp.int32),
            pltpu.VMEM((2, 256, 32), _f32),
            pltpu.VMEM((112, 32), _f32),
            pltpu.VMEM_SHARED((NP, 32), _f32),
            pltpu.SemaphoreType.DMA((2,)),
            pltpu.SemaphoreType.DMA((2,)),
        ],
        compiler_params=_SC_PARAMS,
    )
    def k(r_hbm, dst_hbm, s_hbm, idxb, rb, zb, acc, semi, semsc):
        c = lax.axis_index("c")
        s = lax.axis_index("s")

        @pl.loop(0, 112)
        def _(i):
            for f in range(2):
                zb[i, pl.ds(f * 16, 16)] = jnp.zeros((16,), _f32)

        @pl.loop(0, 28)
        def _(i):
            pltpu.sync_copy(zb, acc.at[pl.ds(s * 3136 + i * 112, 112)])

        plsc.subcore_barrier()

        def in_cps(kk, b):
            return (
                pltpu.make_async_copy(
                    dst_hbm.at[pl.ds(s * 400 + kk * 2, 2)], idxb.at[b],
                    semi.at[b]),
                pltpu.make_async_copy(
                    r_hbm.at[pl.ds(s * 51200 + kk * 256, 256),
                             pl.ds(c * 32, 32)], rb.at[b], semi.at[b]),
            )

        def start_sc(b):
            for j in range(2):
                pltpu.async_copy(rb.at[b].at[pl.ds(j * 128, 128)],
                                 acc.at[idxb.at[b].at[j]], semsc.at[b],
                                 add=True)

        def wait_sc(b):
            for j in range(2):
                pltpu.make_async_copy(rb.at[b].at[pl.ds(j * 128, 128)],
                                      acc.at[idxb.at[b].at[j]],
                                      semsc.at[b]).wait()

        def body(kk, b, first, prefetch):
            for cp in in_cps(0, b):
                cp.wait()
            start_sc(b)
            if not first:
                wait_sc(1 - b)
            if prefetch:
                for cp in in_cps(kk + 1, 1 - b):
                    cp.start()

        for cp in in_cps(0, 0):
            cp.start()
        body(0, 0, True, True)

        @pl.loop(0, 99)
        def _(i):
            body(1 + i * 2, 1, False, True)
            body(2 + i * 2, 0, False, True)

        body(199, 1, False, False)
        wait_sc(1)

        plsc.subcore_barrier()
        pltpu.sync_copy(
            acc.at[pl.ds(s * 3136, 3136)],
            s_hbm.at[pl.ds(s * 3136, 3136), pl.ds(c * 32, 32)],
        )

    return k(r, dst2)


def _h0(xp, batchp, t2, condp, wi1, bi1, wi2, bi2, wt1, bt1, wt2, bt2,
        wc1, bc1, wc2, bc2):
    def body(x_ref, b_ref, t_ref, c_ref, wi1r, bi1r, wi2r, bi2r, wt1r, bt1r,
             wt2r, bt2r, wc1r, bc1r, wc2r, bc2r, o_ref):
        tz = jnp.maximum(t_ref[...] * wt1r[...] + bt1r[...], 0.0)
        tf = jnp.dot(tz, wt2r[...], preferred_element_type=_f32) + bt2r[...]
        cz = jnp.maximum(
            jnp.dot(c_ref[...], wc1r[...], preferred_element_type=_f32)
            + bc1r[...], 0.0)
        cf = jnp.dot(cz, wc2r[...], preferred_element_type=_f32) + bc2r[...]
        tfc = tf + cf
        z = jnp.maximum(
            jnp.dot(x_ref[...], wi1r[...], preferred_element_type=_f32)
            + bi1r[...], 0.0)
        h = jnp.dot(z, wi2r[...], preferred_element_type=_f32) + bi2r[...]
        oh = (b_ref[...] == lax.broadcasted_iota(jnp.int32, (1, B), 1)
              ).astype(_f32)
        o_ref[...] = h + jnp.dot(oh, tfc, preferred_element_type=_f32)

    full = lambda a: pl.BlockSpec(a.shape, lambda i: (0,) * a.ndim)
    return pl.pallas_call(
        body,
        grid=(NP // NBLK,),
        in_specs=[
            pl.BlockSpec((NBLK, 8), lambda i: (i, 0)),
            pl.BlockSpec((NBLK, 1), lambda i: (i, 0)),
            full(t2), full(condp),
            full(wi1), full(bi1), full(wi2), full(bi2),
            full(wt1), full(bt1), full(wt2), full(bt2),
            full(wc1), full(bc1), full(wc2), full(bc2),
        ],
        out_specs=pl.BlockSpec((NBLK, H), lambda i: (i, 0)),
        out_shape=jax.ShapeDtypeStruct((NP, H), _f32),
    )(xp, batchp, t2, condp, wi1, bi1, wi2, bi2, wt1, bt1, wt2, bt2,
      wc1, bc1, wc2, bc2)


def _edge_relu(g, eap, w1r):
    def body(g_ref, ea_ref, w_ref, o_ref):
        o_ref[...] = jnp.maximum(g_ref[...] + ea_ref[...] * w_ref[...], 0.0)

    return pl.pallas_call(
        body,
        grid=(EP // EBLK,),
        in_specs=[
            pl.BlockSpec((EBLK, H), lambda i: (i, 0)),
            pl.BlockSpec((EBLK, 1), lambda i: (i, 0)),
            pl.BlockSpec((1, H), lambda i: (0, 0)),
        ],
        out_specs=pl.BlockSpec((EBLK, H), lambda i: (i, 0)),
        out_shape=jax.ShapeDtypeStruct((EP, H), _f32),
    )(g, eap, w1r)


def _epi_uv(sagg, h, cp0, cp1, w2, b2, a1, b1, a2, want_uv):
    def body(*refs):
        if want_uv:
            (s_ref, h_ref, c0_ref, c1_ref, w2r, b2r, a1r, b1r, a2r,
             ho, uo, vo) = refs
        else:
            s_ref, h_ref, c0_ref, c1_ref, w2r, b2r, ho = refs
        cnt = c0_ref[...] + c1_ref[...]
        invc = 1.0 / jnp.maximum(cnt, 1.0)
        hasb = (cnt > 0.0).astype(_f32)
        q = (jnp.dot(s_ref[...] * invc, w2r[...], preferred_element_type=_f32)
             + b2r[...] * hasb)
        hn = jnp.maximum(q, 0.0) + h_ref[...]
        ho[...] = hn
        if want_uv:
            uo[...] = jnp.dot(hn, a1r[...], preferred_element_type=_f32) + b1r[...]
            vo[...] = jnp.dot(hn, a2r[...], preferred_element_type=_f32)

    nb = pl.BlockSpec((NBLK, H), lambda i: (i, 0))
    cb = pl.BlockSpec((NBLK, 1), lambda i: (i, 0))
    full = lambda a: pl.BlockSpec(a.shape, lambda i: (0,) * a.ndim)
    if want_uv:
        in_specs = [nb, nb, cb, cb, full(w2), full(b2), full(a1), full(b1),
                    full(a2)]
        args = (sagg, h, cp0, cp1, w2, b2, a1, b1, a2)
        out_specs = [nb, nb, nb]
        out_shape = [jax.ShapeDtypeStruct((NP, H), _f32)] * 3
    else:
        in_specs = [nb, nb, cb, cb, full(w2), full(b2)]
        args = (sagg, h, cp0, cp1, w2, b2)
        out_specs = nb
        out_shape = jax.ShapeDtypeStruct((NP, H), _f32)
    return pl.pallas_call(
        body, grid=(NP // NBLK,), in_specs=in_specs, out_specs=out_specs,
        out_shape=out_shape)(*args)


def _uv(h, a1, b1, a2):
    def body(h_ref, a1r, b1r, a2r, uo, vo):
        hn = h_ref[...]
        uo[...] = jnp.dot(hn, a1r[...], preferred_element_type=_f32) + b1r[...]
        vo[...] = jnp.dot(hn, a2r[...], preferred_element_type=_f32)

    nb = pl.BlockSpec((NBLK, H), lambda i: (i, 0))
    full = lambda a: pl.BlockSpec(a.shape, lambda i: (0,) * a.ndim)
    return pl.pallas_call(
        body,
        grid=(NP // NBLK,),
        in_specs=[nb, full(a1), full(b1), full(a2)],
        out_specs=[nb, nb],
        out_shape=[jax.ShapeDtypeStruct((NP, H), _f32)] * 2,
    )(h, a1, b1, a2)


def _out_mlp(h, w1, b1, w2p, b2p):
    def body(h_ref, w1r, b1r, w2r, b2r, o_ref):
        z = jnp.maximum(
            jnp.dot(h_ref[...], w1r[...], preferred_element_type=_f32)
            + b1r[...], 0.0)
        o_ref[...] = jnp.dot(z, w2r[...], preferred_element_type=_f32) + b2r[...]

    nb = pl.BlockSpec((NBLK, H), lambda i: (i, 0))
    full = lambda a: pl.BlockSpec(a.shape, lambda i: (0,) * a.ndim)
    return pl.pallas_call(
        body,
        grid=(NP // NBLK,),
        in_specs=[nb, full(w1), full(b1), full(w2p), full(b2p)],
        out_specs=pl.BlockSpec((NBLK, 8), lambda i: (i, 0)),
        out_shape=jax.ShapeDtypeStruct((NP, 8), _f32),
    )(h, w1, b1, w2p, b2p)


def kernel(x, edge_index, edge_attr, t, batch, condition, params):
    f32 = _f32
    row = lambda b: b.reshape(1, -1).astype(f32)

    src = edge_index[0]
    dst = edge_index[1]
    pad_dst = N + (jnp.arange(EP - E, dtype=jnp.int32) % (NP - N))
    dst2 = jnp.concatenate([dst, pad_dst]).reshape(-1, 128)
    src2 = jnp.concatenate([src, jnp.zeros((EP - E,), jnp.int32)]).reshape(-1, 128)
    eap = jnp.concatenate([edge_attr, jnp.zeros((EP - E, 1), f32)], axis=0)

    xp = jnp.pad(x, ((0, NP - N), (0, 8 - x.shape[1])))
    batchp = jnp.pad(batch, (0, NP - N)).reshape(-1, 1)
    t2 = t.reshape(-1, 1)
    condp = jnp.pad(condition, ((0, 0), (0, 4)))

    ip = params["input_mlp"]
    tp = params["time_mlp"]
    cp = params["cond_mlp"]
    op = params["output_mlp"]

    wi1 = jnp.pad(ip[0]["W"], ((0, 8 - ip[0]["W"].shape[0]), (0, 0)))
    wc1 = jnp.pad(cp[0]["W"], ((0, 4), (0, 0)))
    wo2 = jnp.pad(op[1]["W"], ((0, 0), (0, 8 - op[1]["W"].shape[1])))
    bo2 = jnp.pad(op[1]["b"], (0, 8 - op[1]["b"].shape[0])).reshape(1, -1)

    cnt = _count_edges(dst2)
    cp0 = cnt[0, :, 0:1]
    cp1 = cnt[1, :, 0:1]

    h = _h0(xp, batchp, t2, condp,
            wi1, row(ip[0]["b"]), ip[1]["W"], row(ip[1]["b"]),
            tp[0]["W"], row(tp[0]["b"]), tp[1]["W"], row(tp[1]["b"]),
            wc1, row(cp[0]["b"]), cp[1]["W"], row(cp[1]["b"]))

    convs = params["convs"]
    a1 = convs[0][0]["W"][:H]
    a2 = convs[0][0]["W"][H:2 * H]
    u, v = _uv(h, a1, row(convs[0][0]["b"]), a2)

    for l in range(6):
        w1r = convs[l][0]["W"][2 * H:2 * H + 1]
        g = _gather_uv(u, v, dst2, src2)
        r = _edge_relu(g, eap, w1r)
        sagg = _scatter_add(r, dst2)
        w2 = convs[l][1]["W"]
        b2 = row(convs[l][1]["b"])
        if l < 5:
            na1 = convs[l + 1][0]["W"][:H]
            na2 = convs[l + 1][0]["W"][H:2 * H]
            h, u, v = _epi_uv(sagg, h, cp0, cp1, w2, b2,
                              na1, row(convs[l + 1][0]["b"]), na2, True)
        else:
            h = _epi_uv(sagg, h, cp0, cp1, w2, b2, None, None, None, False)

    outp = _out_mlp(h, op[0]["W"], row(op[0]["b"]), wo2, bo2)
    return outp[:N, :3]


# speedup vs baseline: 2.4571x; 1.0921x over previous
import jax
import jax.numpy as jnp
from jax import lax
from jax.experimental import pallas as pl
from jax.experimental.pallas import tpu as pltpu
from jax.experimental.pallas import tpu_sc as plsc

N = 50000
NP = 50176
E = 800000
EP = 819200
H = 64
B = 16
NBLK = 3136
EBLK = 4096

_f32 = jnp.float32


def _mesh():
    return plsc.VectorSubcoreMesh(core_axis_name="c", subcore_axis_name="s")


_SC_PARAMS = pltpu.CompilerParams(use_tc_tiling_on_sc=False)


def _count_edges(dst2):
    bf16 = jnp.bfloat16

    @pl.kernel(
        out_type=jax.ShapeDtypeStruct((2, NP, 32), bf16),
        mesh=_mesh(),
        scratch_types=[
            pltpu.VMEM((2, 2, 128), jnp.int32),
            pltpu.VMEM((128, 32), bf16),
            pltpu.VMEM((112, 32), bf16),
            pltpu.VMEM_SHARED((NP, 32), bf16),
            pltpu.SemaphoreType.DMA((2,)),
            pltpu.SemaphoreType.DMA((2,)),
        ],
        compiler_params=_SC_PARAMS,
    )
    def k(dst_hbm, cnt_hbm, idxb, ones, zb, acc, semi, semsc):
        c = lax.axis_index("c")
        s = lax.axis_index("s")
        row0 = c * 3200 + s * 200

        @pl.loop(0, 112)
        def _(i):
            zb[i, pl.ds(0, 32)] = jnp.zeros((32,), bf16)

        @pl.loop(0, 128)
        def _(i):
            ones[i, pl.ds(0, 32)] = jnp.ones((32,), bf16)

        @pl.loop(0, 28)
        def _(i):
            pltpu.sync_copy(zb, acc.at[pl.ds(s * 3136 + i * 112, 112)])

        plsc.subcore_barrier()

        def in_cp(kk, b):
            return pltpu.make_async_copy(
                dst_hbm.at[pl.ds(row0 + kk * 2, 2)], idxb.at[b], semi.at[b])

        def start_sc(b):
            for j in range(2):
                pltpu.async_copy(ones, acc.at[idxb.at[b].at[j]],
                                 semsc.at[b], add=True)

        def wait_sc(b):
            for j in range(2):
                pltpu.make_async_copy(ones, acc.at[idxb.at[b].at[j]],
                                      semsc.at[b]).wait()

        def body(kk, b, first, prefetch):
            in_cp(0, b).wait()
            start_sc(b)
            if not first:
                wait_sc(1 - b)
            if prefetch:
                in_cp(kk + 1, 1 - b).start()

        in_cp(0, 0).start()
        body(0, 0, True, True)

        @pl.loop(0, 49)
        def _(i):
            body(1 + i * 2, 1, False, True)
            body(2 + i * 2, 0, False, True)

        body(99, 1, False, False)
        wait_sc(1)

        plsc.subcore_barrier()
        pltpu.sync_copy(
            acc.at[pl.ds(s * 3136, 3136)],
            cnt_hbm.at[c, pl.ds(s * 3136, 3136)],
        )

    return k(dst2)


def _gather_uv(u, v, dst2, src2):
    bf16 = jnp.bfloat16

    @pl.kernel(
        out_type=jax.ShapeDtypeStruct((EP, H), bf16),
        mesh=_mesh(),
        scratch_types=[
            pltpu.VMEM((2, 2, 128), jnp.int32),
            pltpu.VMEM((2, 2, 128), jnp.int32),
            pltpu.VMEM((2, 256, H), bf16),
            pltpu.VMEM((2, 256, H), bf16),
            pltpu.SemaphoreType.DMA((2,)),
            pltpu.SemaphoreType.DMA((2,)),
            pltpu.SemaphoreType.DMA((2,)),
        ],
        compiler_params=_SC_PARAMS,
    )
    def k(u_hbm, v_hbm, dst_hbm, src_hbm, g_hbm, db, sb, ub, vb,
          semi, semg, semo):
        c = lax.axis_index("c")
        s = lax.axis_index("s")
        wid = s * 2 + c
        row0 = wid * 200
        e0 = wid * 25600

        def idx_cps(kk, b):
            r = row0 + kk * 2
            return (
                pltpu.make_async_copy(dst_hbm.at[pl.ds(r, 2)], db.at[b],
                                      semi.at[b]),
                pltpu.make_async_copy(src_hbm.at[pl.ds(r, 2)], sb.at[b],
                                      semi.at[b]),
            )

        def g_cps(b):
            cps = []
            for j in range(2):
                cps.append(pltpu.make_async_copy(
                    u_hbm.at[db.at[b].at[j]],
                    ub.at[b].at[pl.ds(j * 128, 128)], semg.at[b]))
                cps.append(pltpu.make_async_copy(
                    v_hbm.at[sb.at[b].at[j]],
                    vb.at[b].at[pl.ds(j * 128, 128)], semg.at[b]))
            return cps

        def out_cp(kk, b):
            return pltpu.make_async_copy(
                ub.at[b], g_hbm.at[pl.ds(e0 + kk * 256, 256)], semo.at[b])

        def finish(kk, b):
            for cp in g_cps(b):
                cp.wait()

            @pl.loop(0, 256, step=8)
            def _(r):
                for rr in range(8):
                    for f in range(2):
                        sl = pl.ds(f * 32, 32)
                        ub[b, r + rr, sl] = ub[b, r + rr, sl] + vb[b, r + rr, sl]

            out_cp(kk, b).start()

        def it(kk, b, wait_o, prefetch, fin_prev):
            for cp in idx_cps(0, b):
                cp.wait()
            if wait_o:
                out_cp(0, b).wait()
            for cp in g_cps(b):
                cp.start()
            if fin_prev:
                finish(kk - 1, 1 - b)
            if prefetch:
                for cp in idx_cps(kk + 1, 1 - b):
                    cp.start()

        for cp in idx_cps(0, 0):
            cp.start()
        it(0, 0, False, True, False)
        it(1, 1, False, True, True)
        it(2, 0, True, True, True)

        @pl.loop(0, 48)
        def _(i):
            it(3 + i * 2, 1, True, True, True)
            it(4 + i * 2, 0, True, True, True)

        it(99, 1, True, False, True)
        finish(99, 1)
        out_cp(0, 0).wait()
        out_cp(0, 1).wait()

    return k(u, v, dst2, src2)


def _scatter_add(r, dst2):
    bf16 = jnp.bfloat16

    @pl.kernel(
        out_type=jax.ShapeDtypeStruct((NP, H), bf16),
        mesh=_mesh(),
        scratch_types=[
            pltpu.VMEM((2, 2, 128), jnp.int32),
            pltpu.VMEM((2, 256, 32), bf16),
            pltpu.VMEM((112, 32), bf16),
            pltpu.VMEM_SHARED((NP, 32), bf16),
            pltpu.SemaphoreType.DMA((2,)),
            pltpu.SemaphoreType.DMA((2,)),
        ],
        compiler_params=_SC_PARAMS,
    )
    def k(r_hbm, dst_hbm, s_hbm, idxb, rb, zb, acc, semi, semsc):
        c = lax.axis_index("c")
        s = lax.axis_index("s")

        @pl.loop(0, 112)
        def _(i):
            zb[i, pl.ds(0, 32)] = jnp.zeros((32,), bf16)

        @pl.loop(0, 28)
        def _(i):
            pltpu.sync_copy(zb, acc.at[pl.ds(s * 3136 + i * 112, 112)])

        plsc.subcore_barrier()

        def in_cps(kk, b):
            return (
                pltpu.make_async_copy(
                    dst_hbm.at[pl.ds(s * 400 + kk * 2, 2)], idxb.at[b],
                    semi.at[b]),
                pltpu.make_async_copy(
                    r_hbm.at[pl.ds(s * 51200 + kk * 256, 256),
                             pl.ds(c * 32, 32)], rb.at[b], semi.at[b]),
            )

        def start_sc(b):
            for j in range(2):
                pltpu.async_copy(rb.at[b].at[pl.ds(j * 128, 128)],
                                 acc.at[idxb.at[b].at[j]], semsc.at[b],
                                 add=True)

        def wait_sc(b):
            for j in range(2):
                pltpu.make_async_copy(rb.at[b].at[pl.ds(j * 128, 128)],
                                      acc.at[idxb.at[b].at[j]],
                                      semsc.at[b]).wait()

        def body(kk, b, first, prefetch):
            for cp in in_cps(0, b):
                cp.wait()
            start_sc(b)
            if not first:
                wait_sc(1 - b)
            if prefetch:
                for cp in in_cps(kk + 1, 1 - b):
                    cp.start()

        for cp in in_cps(0, 0):
            cp.start()
        body(0, 0, True, True)

        @pl.loop(0, 99)
        def _(i):
            body(1 + i * 2, 1, False, True)
            body(2 + i * 2, 0, False, True)

        body(199, 1, False, False)
        wait_sc(1)

        plsc.subcore_barrier()
        pltpu.sync_copy(
            acc.at[pl.ds(s * 3136, 3136)],
            s_hbm.at[pl.ds(s * 3136, 3136), pl.ds(c * 32, 32)],
        )

    return k(r, dst2)


def _h0(xp, batchp, t2, condp, wi1, bi1, wi2, bi2, wt1, bt1, wt2, bt2,
        wc1, bc1, wc2, bc2):
    def body(x_ref, b_ref, t_ref, c_ref, wi1r, bi1r, wi2r, bi2r, wt1r, bt1r,
             wt2r, bt2r, wc1r, bc1r, wc2r, bc2r, o_ref):
        tz = jnp.maximum(t_ref[...] * wt1r[...] + bt1r[...], 0.0)
        tf = jnp.dot(tz, wt2r[...], preferred_element_type=_f32) + bt2r[...]
        cz = jnp.maximum(
            jnp.dot(c_ref[...], wc1r[...], preferred_element_type=_f32)
            + bc1r[...], 0.0)
        cf = jnp.dot(cz, wc2r[...], preferred_element_type=_f32) + bc2r[...]
        tfc = tf + cf
        z = jnp.maximum(
            jnp.dot(x_ref[...], wi1r[...], preferred_element_type=_f32)
            + bi1r[...], 0.0)
        h = jnp.dot(z, wi2r[...], preferred_element_type=_f32) + bi2r[...]
        oh = (b_ref[...] == lax.broadcasted_iota(jnp.int32, (1, B), 1)
              ).astype(_f32)
        o_ref[...] = h + jnp.dot(oh, tfc, preferred_element_type=_f32)

    full = lambda a: pl.BlockSpec(a.shape, lambda i: (0,) * a.ndim)
    return pl.pallas_call(
        body,
        grid=(NP // NBLK,),
        in_specs=[
            pl.BlockSpec((NBLK, 8), lambda i: (i, 0)),
            pl.BlockSpec((NBLK, 1), lambda i: (i, 0)),
            full(t2), full(condp),
            full(wi1), full(bi1), full(wi2), full(bi2),
            full(wt1), full(bt1), full(wt2), full(bt2),
            full(wc1), full(bc1), full(wc2), full(bc2),
        ],
        out_specs=pl.BlockSpec((NBLK, H), lambda i: (i, 0)),
        out_shape=jax.ShapeDtypeStruct((NP, H), _f32),
    )(xp, batchp, t2, condp, wi1, bi1, wi2, bi2, wt1, bt1, wt2, bt2,
      wc1, bc1, wc2, bc2)


def _edge_relu(g, eap, w1r):
    def body(g_ref, ea_ref, w_ref, o_ref):
        o_ref[...] = jnp.maximum(
            g_ref[...].astype(_f32) + ea_ref[...] * w_ref[...],
            0.0).astype(jnp.bfloat16)

    return pl.pallas_call(
        body,
        grid=(EP // EBLK,),
        in_specs=[
            pl.BlockSpec((EBLK, H), lambda i: (i, 0)),
            pl.BlockSpec((EBLK, 1), lambda i: (i, 0)),
            pl.BlockSpec((1, H), lambda i: (0, 0)),
        ],
        out_specs=pl.BlockSpec((EBLK, H), lambda i: (i, 0)),
        out_shape=jax.ShapeDtypeStruct((EP, H), jnp.bfloat16),
    )(g, eap, w1r)


def _epi_uv(sagg, h, cp0, cp1, w2, b2, a1, b1, a2, want_uv):
    def body(*refs):
        if want_uv:
            (s_ref, h_ref, c0_ref, c1_ref, w2r, b2r, a1r, b1r, a2r,
             ho, uo, vo) = refs
        else:
            s_ref, h_ref, c0_ref, c1_ref, w2r, b2r, ho = refs
        cnt = c0_ref[...].astype(_f32) + c1_ref[...].astype(_f32)
        invc = 1.0 / jnp.maximum(cnt, 1.0)
        hasb = (cnt > 0.0).astype(_f32)
        q = (jnp.dot(s_ref[...].astype(_f32) * invc, w2r[...],
                     preferred_element_type=_f32)
             + b2r[...] * hasb)
        hn = jnp.maximum(q, 0.0) + h_ref[...]
        ho[...] = hn
        if want_uv:
            uo[...] = (jnp.dot(hn, a1r[...], preferred_element_type=_f32)
                       + b1r[...]).astype(jnp.bfloat16)
            vo[...] = jnp.dot(hn, a2r[...],
                              preferred_element_type=_f32).astype(jnp.bfloat16)

    nb = pl.BlockSpec((NBLK, H), lambda i: (i, 0))
    cb = pl.BlockSpec((NBLK, 1), lambda i: (i, 0))
    full = lambda a: pl.BlockSpec(a.shape, lambda i: (0,) * a.ndim)
    if want_uv:
        in_specs = [nb, nb, cb, cb, full(w2), full(b2), full(a1), full(b1),
                    full(a2)]
        args = (sagg, h, cp0, cp1, w2, b2, a1, b1, a2)
        out_specs = [nb, nb, nb]
        out_shape = [jax.ShapeDtypeStruct((NP, H), _f32),
                     jax.ShapeDtypeStruct((NP, H), jnp.bfloat16),
                     jax.ShapeDtypeStruct((NP, H), jnp.bfloat16)]
    else:
        in_specs = [nb, nb, cb, cb, full(w2), full(b2)]
        args = (sagg, h, cp0, cp1, w2, b2)
        out_specs = nb
        out_shape = jax.ShapeDtypeStruct((NP, H), _f32)
    return pl.pallas_call(
        body, grid=(NP // NBLK,), in_specs=in_specs, out_specs=out_specs,
        out_shape=out_shape)(*args)


def _uv(h, a1, b1, a2):
    def body(h_ref, a1r, b1r, a2r, uo, vo):
        hn = h_ref[...]
        uo[...] = (jnp.dot(hn, a1r[...], preferred_element_type=_f32)
                   + b1r[...]).astype(jnp.bfloat16)
        vo[...] = jnp.dot(hn, a2r[...],
                          preferred_element_type=_f32).astype(jnp.bfloat16)

    nb = pl.BlockSpec((NBLK, H), lambda i: (i, 0))
    full = lambda a: pl.BlockSpec(a.shape, lambda i: (0,) * a.ndim)
    return pl.pallas_call(
        body,
        grid=(NP // NBLK,),
        in_specs=[nb, full(a1), full(b1), full(a2)],
        out_specs=[nb, nb],
        out_shape=[jax.ShapeDtypeStruct((NP, H), jnp.bfloat16)] * 2,
    )(h, a1, b1, a2)


def _out_mlp(h, w1, b1, w2p, b2p):
    def body(h_ref, w1r, b1r, w2r, b2r, o_ref):
        z = jnp.maximum(
            jnp.dot(h_ref[...], w1r[...], preferred_element_type=_f32)
            + b1r[...], 0.0)
        o_ref[...] = jnp.dot(z, w2r[...], preferred_element_type=_f32) + b2r[...]

    nb = pl.BlockSpec((NBLK, H), lambda i: (i, 0))
    full = lambda a: pl.BlockSpec(a.shape, lambda i: (0,) * a.ndim)
    return pl.pallas_call(
        body,
        grid=(NP // NBLK,),
        in_specs=[nb, full(w1), full(b1), full(w2p), full(b2p)],
        out_specs=pl.BlockSpec((NBLK, 8), lambda i: (i, 0)),
        out_shape=jax.ShapeDtypeStruct((NP, 8), _f32),
    )(h, w1, b1, w2p, b2p)


def kernel(x, edge_index, edge_attr, t, batch, condition, params):
    f32 = _f32
    row = lambda b: b.reshape(1, -1).astype(f32)

    src = edge_index[0]
    dst = edge_index[1]
    pad_dst = N + (jnp.arange(EP - E, dtype=jnp.int32) % (NP - N))
    dst2 = jnp.concatenate([dst, pad_dst]).reshape(-1, 128)
    src2 = jnp.concatenate([src, jnp.zeros((EP - E,), jnp.int32)]).reshape(-1, 128)
    eap = jnp.concatenate([edge_attr, jnp.zeros((EP - E, 1), f32)], axis=0)

    xp = jnp.pad(x, ((0, NP - N), (0, 8 - x.shape[1])))
    batchp = jnp.pad(batch, (0, NP - N)).reshape(-1, 1)
    t2 = t.reshape(-1, 1)
    condp = jnp.pad(condition, ((0, 0), (0, 4)))

    ip = params["input_mlp"]
    tp = params["time_mlp"]
    cp = params["cond_mlp"]
    op = params["output_mlp"]

    wi1 = jnp.pad(ip[0]["W"], ((0, 8 - ip[0]["W"].shape[0]), (0, 0)))
    wc1 = jnp.pad(cp[0]["W"], ((0, 4), (0, 0)))
    wo2 = jnp.pad(op[1]["W"], ((0, 0), (0, 8 - op[1]["W"].shape[1])))
    bo2 = jnp.pad(op[1]["b"], (0, 8 - op[1]["b"].shape[0])).reshape(1, -1)

    cnt = _count_edges(dst2)
    cp0 = cnt[0, :, 0:1]
    cp1 = cnt[1, :, 0:1]

    h = _h0(xp, batchp, t2, condp,
            wi1, row(ip[0]["b"]), ip[1]["W"], row(ip[1]["b"]),
            tp[0]["W"], row(tp[0]["b"]), tp[1]["W"], row(tp[1]["b"]),
            wc1, row(cp[0]["b"]), cp[1]["W"], row(cp[1]["b"]))

    convs = params["convs"]
    a1 = convs[0][0]["W"][:H]
    a2 = convs[0][0]["W"][H:2 * H]
    u, v = _uv(h, a1, row(convs[0][0]["b"]), a2)

    for l in range(6):
        w1r = convs[l][0]["W"][2 * H:2 * H + 1]
        g = _gather_uv(u, v, dst2, src2)
        r = _edge_relu(g, eap, w1r)
        sagg = _scatter_add(r, dst2)
        w2 = convs[l][1]["W"]
        b2 = row(convs[l][1]["b"])
        if l < 5:
            na1 = convs[l + 1][0]["W"][:H]
            na2 = convs[l + 1][0]["W"][H:2 * H]
            h, u, v = _epi_uv(sagg, h, cp0, cp1, w2, b2,
                              na1, row(convs[l + 1][0]["b"]), na2, True)
        else:
            h = _epi_uv(sagg, h, cp0, cp1, w2, b2, None, None, None, False)

    outp = _out_mlp(h, op[0]["W"], row(op[0]["b"]), wo2, bo2)
    return outp[:N, :3]


# speedup vs baseline: 2.4748x; 1.0072x over previous
import jax
import jax.numpy as jnp
from jax import lax
from jax.experimental import pallas as pl
from jax.experimental.pallas import tpu as pltpu
from jax.experimental.pallas import tpu_sc as plsc

N = 50000
NP = 50176
E = 800000
EP = 819200
H = 64
B = 16
NBLK = 3136
EBLK = 4096

_f32 = jnp.float32


def _mesh():
    return plsc.VectorSubcoreMesh(core_axis_name="c", subcore_axis_name="s")


_SC_PARAMS = pltpu.CompilerParams(use_tc_tiling_on_sc=False)


def _count_edges(dst2):
    bf16 = jnp.bfloat16

    @pl.kernel(
        out_type=jax.ShapeDtypeStruct((2, NP, 32), bf16),
        mesh=_mesh(),
        scratch_types=[
            pltpu.VMEM((4, 2, 128), jnp.int32),
            pltpu.VMEM((128, 32), bf16),
            pltpu.VMEM((112, 32), bf16),
            pltpu.VMEM_SHARED((NP, 32), bf16),
            pltpu.SemaphoreType.DMA((4,)),
            pltpu.SemaphoreType.DMA((4,)),
        ],
        compiler_params=_SC_PARAMS,
    )
    def k(dst_hbm, cnt_hbm, idxb, ones, zb, acc, semi, semsc):
        c = lax.axis_index("c")
        s = lax.axis_index("s")
        row0 = c * 3200 + s * 200

        @pl.loop(0, 112)
        def _(i):
            zb[i, pl.ds(0, 32)] = jnp.zeros((32,), bf16)

        @pl.loop(0, 128)
        def _(i):
            ones[i, pl.ds(0, 32)] = jnp.ones((32,), bf16)

        @pl.loop(0, 28)
        def _(i):
            pltpu.sync_copy(zb, acc.at[pl.ds(s * 3136 + i * 112, 112)])

        plsc.subcore_barrier()

        def in_cp(kk, b):
            return pltpu.make_async_copy(
                dst_hbm.at[pl.ds(row0 + kk * 2, 2)], idxb.at[b], semi.at[b])

        def start_sc(b):
            for j in range(2):
                pltpu.async_copy(ones, acc.at[idxb.at[b].at[j]],
                                 semsc.at[b], add=True)

        def wait_sc(b):
            for j in range(2):
                pltpu.make_async_copy(ones, acc.at[idxb.at[b].at[j]],
                                      semsc.at[b]).wait()

        def body(kk, b, first, prefetch):
            in_cp(0, b).wait()
            start_sc(b)
            if not first:
                wait_sc((b + 1) % 4)
            if prefetch:
                in_cp(kk + 1, (b + 1) % 4).start()

        in_cp(0, 0).start()
        body(0, 0, True, True)
        body(1, 1, True, True)
        body(2, 2, True, True)

        @pl.loop(0, 24)
        def _(i):
            for q in range(4):
                body(3 + i * 4 + q, (3 + q) % 4, False, True)

        body(99, 3, False, False)
        wait_sc(1)
        wait_sc(2)
        wait_sc(3)

        plsc.subcore_barrier()
        pltpu.sync_copy(
            acc.at[pl.ds(s * 3136, 3136)],
            cnt_hbm.at[c, pl.ds(s * 3136, 3136)],
        )

    return k(dst2)


def _gather_uv(u, v, dst2, src2):
    bf16 = jnp.bfloat16

    @pl.kernel(
        out_type=jax.ShapeDtypeStruct((EP, H), bf16),
        mesh=_mesh(),
        scratch_types=[
            pltpu.VMEM((4, 2, 128), jnp.int32),
            pltpu.VMEM((4, 2, 128), jnp.int32),
            pltpu.VMEM((4, 256, H), bf16),
            pltpu.VMEM((4, 256, H), bf16),
            pltpu.SemaphoreType.DMA((4,)),
            pltpu.SemaphoreType.DMA((4,)),
            pltpu.SemaphoreType.DMA((4,)),
        ],
        compiler_params=_SC_PARAMS,
    )
    def k(u_hbm, v_hbm, dst_hbm, src_hbm, g_hbm, db, sb, ub, vb,
          semi, semg, semo):
        c = lax.axis_index("c")
        s = lax.axis_index("s")
        wid = s * 2 + c
        row0 = wid * 200
        e0 = wid * 25600

        def idx_cps(kk, b):
            r = row0 + kk * 2
            return (
                pltpu.make_async_copy(dst_hbm.at[pl.ds(r, 2)], db.at[b],
                                      semi.at[b]),
                pltpu.make_async_copy(src_hbm.at[pl.ds(r, 2)], sb.at[b],
                                      semi.at[b]),
            )

        def g_cps(b):
            cps = []
            for j in range(2):
                cps.append(pltpu.make_async_copy(
                    u_hbm.at[db.at[b].at[j]],
                    ub.at[b].at[pl.ds(j * 128, 128)], semg.at[b]))
                cps.append(pltpu.make_async_copy(
                    v_hbm.at[sb.at[b].at[j]],
                    vb.at[b].at[pl.ds(j * 128, 128)], semg.at[b]))
            return cps

        def out_cp(kk, b):
            return pltpu.make_async_copy(
                ub.at[b], g_hbm.at[pl.ds(e0 + kk * 256, 256)], semo.at[b])

        def finish(kk, b):
            for cp in g_cps(b):
                cp.wait()

            @pl.loop(0, 256, step=8)
            def _(r):
                for rr in range(8):
                    for f in range(2):
                        sl = pl.ds(f * 32, 32)
                        ub[b, r + rr, sl] = ub[b, r + rr, sl] + vb[b, r + rr, sl]

            out_cp(kk, b).start()

        def it(kk, b, wait_o, prefetch, fin_prev):
            for cp in idx_cps(0, b):
                cp.wait()
            if wait_o:
                out_cp(0, b).wait()
            for cp in g_cps(b):
                cp.start()
            if fin_prev:
                finish(kk - 3, (b + 1) % 4)
            if prefetch:
                for cp in idx_cps(kk + 1, (b + 1) % 4):
                    cp.start()

        for cp in idx_cps(0, 0):
            cp.start()
        it(0, 0, False, True, False)
        it(1, 1, False, True, False)
        it(2, 2, False, True, False)
        it(3, 3, False, True, True)

        @pl.loop(0, 23)
        def _(i):
            for q in range(4):
                it(4 + i * 4 + q, q, True, True, True)

        it(96, 0, True, True, True)
        it(97, 1, True, True, True)
        it(98, 2, True, True, True)
        it(99, 3, True, False, True)
        finish(97, 1)
        finish(98, 2)
        finish(99, 3)
        for b in range(4):
            out_cp(0, b).wait()

    return k(u, v, dst2, src2)


def _scatter_add(r, dst2):
    bf16 = jnp.bfloat16

    @pl.kernel(
        out_type=jax.ShapeDtypeStruct((NP, H), bf16),
        mesh=_mesh(),
        scratch_types=[
            pltpu.VMEM((4, 2, 128), jnp.int32),
            pltpu.VMEM((4, 256, 32), bf16),
            pltpu.VMEM((112, 32), bf16),
            pltpu.VMEM_SHARED((NP, 32), bf16),
            pltpu.SemaphoreType.DMA((4,)),
            pltpu.SemaphoreType.DMA((4,)),
        ],
        compiler_params=_SC_PARAMS,
    )
    def k(r_hbm, dst_hbm, s_hbm, idxb, rb, zb, acc, semi, semsc):
        c = lax.axis_index("c")
        s = lax.axis_index("s")

        @pl.loop(0, 112)
        def _(i):
            zb[i, pl.ds(0, 32)] = jnp.zeros((32,), bf16)

        @pl.loop(0, 28)
        def _(i):
            pltpu.sync_copy(zb, acc.at[pl.ds(s * 3136 + i * 112, 112)])

        plsc.subcore_barrier()

        def in_cps(kk, b):
            return (
                pltpu.make_async_copy(
                    dst_hbm.at[pl.ds(s * 400 + kk * 2, 2)], idxb.at[b],
                    semi.at[b]),
                pltpu.make_async_copy(
                    r_hbm.at[pl.ds(s * 51200 + kk * 256, 256),
                             pl.ds(c * 32, 32)], rb.at[b], semi.at[b]),
            )

        def start_sc(b):
            for j in range(2):
                pltpu.async_copy(rb.at[b].at[pl.ds(j * 128, 128)],
                                 acc.at[idxb.at[b].at[j]], semsc.at[b],
                                 add=True)

        def wait_sc(b):
            for j in range(2):
                pltpu.make_async_copy(rb.at[b].at[pl.ds(j * 128, 128)],
                                      acc.at[idxb.at[b].at[j]],
                                      semsc.at[b]).wait()

        def body(kk, b, first, prefetch):
            for cp in in_cps(0, b):
                cp.wait()
            start_sc(b)
            if not first:
                wait_sc((b + 1) % 4)
            if prefetch:
                for cp in in_cps(kk + 1, (b + 1) % 4):
                    cp.start()

        for cp in in_cps(0, 0):
            cp.start()
        body(0, 0, True, True)
        body(1, 1, True, True)
        body(2, 2, True, True)

        @pl.loop(0, 49)
        def _(i):
            for q in range(4):
                body(3 + i * 4 + q, (3 + q) % 4, False, True)

        body(199, 3, False, False)
        wait_sc(1)
        wait_sc(2)
        wait_sc(3)

        plsc.subcore_barrier()
        pltpu.sync_copy(
            acc.at[pl.ds(s * 3136, 3136)],
            s_hbm.at[pl.ds(s * 3136, 3136), pl.ds(c * 32, 32)],
        )

    return k(r, dst2)


def _h0(xp, batchp, t2, condp, wi1, bi1, wi2, bi2, wt1, bt1, wt2, bt2,
        wc1, bc1, wc2, bc2):
    def body(x_ref, b_ref, t_ref, c_ref, wi1r, bi1r, wi2r, bi2r, wt1r, bt1r,
             wt2r, bt2r, wc1r, bc1r, wc2r, bc2r, o_ref):
        tz = jnp.maximum(t_ref[...] * wt1r[...] + bt1r[...], 0.0)
        tf = jnp.dot(tz, wt2r[...], preferred_element_type=_f32) + bt2r[...]
        cz = jnp.maximum(
            jnp.dot(c_ref[...], wc1r[...], preferred_element_type=_f32)
            + bc1r[...], 0.0)
        cf = jnp.dot(cz, wc2r[...], preferred_element_type=_f32) + bc2r[...]
        tfc = tf + cf
        z = jnp.maximum(
            jnp.dot(x_ref[...], wi1r[...], preferred_element_type=_f32)
            + bi1r[...], 0.0)
        h = jnp.dot(z, wi2r[...], preferred_element_type=_f32) + bi2r[...]
        oh = (b_ref[...] == lax.broadcasted_iota(jnp.int32, (1, B), 1)
              ).astype(_f32)
        o_ref[...] = h + jnp.dot(oh, tfc, preferred_element_type=_f32)

    full = lambda a: pl.BlockSpec(a.shape, lambda i: (0,) * a.ndim)
    return pl.pallas_call(
        body,
        grid=(NP // NBLK,),
        in_specs=[
            pl.BlockSpec((NBLK, 8), lambda i: (i, 0)),
            pl.BlockSpec((NBLK, 1), lambda i: (i, 0)),
            full(t2), full(condp),
            full(wi1), full(bi1), full(wi2), full(bi2),
            full(wt1), full(bt1), full(wt2), full(bt2),
            full(wc1), full(bc1), full(wc2), full(bc2),
        ],
        out_specs=pl.BlockSpec((NBLK, H), lambda i: (i, 0)),
        out_shape=jax.ShapeDtypeStruct((NP, H), _f32),
    )(xp, batchp, t2, condp, wi1, bi1, wi2, bi2, wt1, bt1, wt2, bt2,
      wc1, bc1, wc2, bc2)


def _edge_relu(g, eap, w1r):
    def body(g_ref, ea_ref, w_ref, o_ref):
        o_ref[...] = jnp.maximum(
            g_ref[...].astype(_f32) + ea_ref[...] * w_ref[...],
            0.0).astype(jnp.bfloat16)

    return pl.pallas_call(
        body,
        grid=(EP // EBLK,),
        in_specs=[
            pl.BlockSpec((EBLK, H), lambda i: (i, 0)),
            pl.BlockSpec((EBLK, 1), lambda i: (i, 0)),
            pl.BlockSpec((1, H), lambda i: (0, 0)),
        ],
        out_specs=pl.BlockSpec((EBLK, H), lambda i: (i, 0)),
        out_shape=jax.ShapeDtypeStruct((EP, H), jnp.bfloat16),
    )(g, eap, w1r)


def _epi_uv(sagg, h, cp0, cp1, w2, b2, a1, b1, a2, want_uv):
    def body(*refs):
        if want_uv:
            (s_ref, h_ref, c0_ref, c1_ref, w2r, b2r, a1r, b1r, a2r,
             ho, uo, vo) = refs
        else:
            s_ref, h_ref, c0_ref, c1_ref, w2r, b2r, ho = refs
        cnt = c0_ref[...].astype(_f32) + c1_ref[...].astype(_f32)
        invc = 1.0 / jnp.maximum(cnt, 1.0)
        hasb = (cnt > 0.0).astype(_f32)
        q = (jnp.dot(s_ref[...].astype(_f32) * invc, w2r[...],
                     preferred_element_type=_f32)
             + b2r[...] * hasb)
        hn = jnp.maximum(q, 0.0) + h_ref[...]
        ho[...] = hn
        if want_uv:
            uo[...] = (jnp.dot(hn, a1r[...], preferred_element_type=_f32)
                       + b1r[...]).astype(jnp.bfloat16)
            vo[...] = jnp.dot(hn, a2r[...],
                              preferred_element_type=_f32).astype(jnp.bfloat16)

    nb = pl.BlockSpec((NBLK, H), lambda i: (i, 0))
    cb = pl.BlockSpec((NBLK, 1), lambda i: (i, 0))
    full = lambda a: pl.BlockSpec(a.shape, lambda i: (0,) * a.ndim)
    if want_uv:
        in_specs = [nb, nb, cb, cb, full(w2), full(b2), full(a1), full(b1),
                    full(a2)]
        args = (sagg, h, cp0, cp1, w2, b2, a1, b1, a2)
        out_specs = [nb, nb, nb]
        out_shape = [jax.ShapeDtypeStruct((NP, H), _f32),
                     jax.ShapeDtypeStruct((NP, H), jnp.bfloat16),
                     jax.ShapeDtypeStruct((NP, H), jnp.bfloat16)]
    else:
        in_specs = [nb, nb, cb, cb, full(w2), full(b2)]
        args = (sagg, h, cp0, cp1, w2, b2)
        out_specs = nb
        out_shape = jax.ShapeDtypeStruct((NP, H), _f32)
    return pl.pallas_call(
        body, grid=(NP // NBLK,), in_specs=in_specs, out_specs=out_specs,
        out_shape=out_shape)(*args)


def _uv(h, a1, b1, a2):
    def body(h_ref, a1r, b1r, a2r, uo, vo):
        hn = h_ref[...]
        uo[...] = (jnp.dot(hn, a1r[...], preferred_element_type=_f32)
                   + b1r[...]).astype(jnp.bfloat16)
        vo[...] = jnp.dot(hn, a2r[...],
                          preferred_element_type=_f32).astype(jnp.bfloat16)

    nb = pl.BlockSpec((NBLK, H), lambda i: (i, 0))
    full = lambda a: pl.BlockSpec(a.shape, lambda i: (0,) * a.ndim)
    return pl.pallas_call(
        body,
        grid=(NP // NBLK,),
        in_specs=[nb, full(a1), full(b1), full(a2)],
        out_specs=[nb, nb],
        out_shape=[jax.ShapeDtypeStruct((NP, H), jnp.bfloat16)] * 2,
    )(h, a1, b1, a2)


def _out_mlp(h, w1, b1, w2p, b2p):
    def body(h_ref, w1r, b1r, w2r, b2r, o_ref):
        z = jnp.maximum(
            jnp.dot(h_ref[...], w1r[...], preferred_element_type=_f32)
            + b1r[...], 0.0)
        o_ref[...] = jnp.dot(z, w2r[...], preferred_element_type=_f32) + b2r[...]

    nb = pl.BlockSpec((NBLK, H), lambda i: (i, 0))
    full = lambda a: pl.BlockSpec(a.shape, lambda i: (0,) * a.ndim)
    return pl.pallas_call(
        body,
        grid=(NP // NBLK,),
        in_specs=[nb, full(w1), full(b1), full(w2p), full(b2p)],
        out_specs=pl.BlockSpec((NBLK, 8), lambda i: (i, 0)),
        out_shape=jax.ShapeDtypeStruct((NP, 8), _f32),
    )(h, w1, b1, w2p, b2p)


def kernel(x, edge_index, edge_attr, t, batch, condition, params):
    f32 = _f32
    row = lambda b: b.reshape(1, -1).astype(f32)

    src = edge_index[0]
    dst = edge_index[1]
    pad_dst = N + (jnp.arange(EP - E, dtype=jnp.int32) % (NP - N))
    dst2 = jnp.concatenate([dst, pad_dst]).reshape(-1, 128)
    src2 = jnp.concatenate([src, jnp.zeros((EP - E,), jnp.int32)]).reshape(-1, 128)
    eap = jnp.concatenate([edge_attr, jnp.zeros((EP - E, 1), f32)], axis=0)

    xp = jnp.pad(x, ((0, NP - N), (0, 8 - x.shape[1])))
    batchp = jnp.pad(batch, (0, NP - N)).reshape(-1, 1)
    t2 = t.reshape(-1, 1)
    condp = jnp.pad(condition, ((0, 0), (0, 4)))

    ip = params["input_mlp"]
    tp = params["time_mlp"]
    cp = params["cond_mlp"]
    op = params["output_mlp"]

    wi1 = jnp.pad(ip[0]["W"], ((0, 8 - ip[0]["W"].shape[0]), (0, 0)))
    wc1 = jnp.pad(cp[0]["W"], ((0, 4), (0, 0)))
    wo2 = jnp.pad(op[1]["W"], ((0, 0), (0, 8 - op[1]["W"].shape[1])))
    bo2 = jnp.pad(op[1]["b"], (0, 8 - op[1]["b"].shape[0])).reshape(1, -1)

    cnt = _count_edges(dst2)
    cp0 = cnt[0, :, 0:1]
    cp1 = cnt[1, :, 0:1]

    h = _h0(xp, batchp, t2, condp,
            wi1, row(ip[0]["b"]), ip[1]["W"], row(ip[1]["b"]),
            tp[0]["W"], row(tp[0]["b"]), tp[1]["W"], row(tp[1]["b"]),
            wc1, row(cp[0]["b"]), cp[1]["W"], row(cp[1]["b"]))

    convs = params["convs"]
    a1 = convs[0][0]["W"][:H]
    a2 = convs[0][0]["W"][H:2 * H]
    u, v = _uv(h, a1, row(convs[0][0]["b"]), a2)

    for l in range(6):
        w1r = convs[l][0]["W"][2 * H:2 * H + 1]
        g = _gather_uv(u, v, dst2, src2)
        r = _edge_relu(g, eap, w1r)
        sagg = _scatter_add(r, dst2)
        w2 = convs[l][1]["W"]
        b2 = row(convs[l][1]["b"])
        if l < 5:
            na1 = convs[l + 1][0]["W"][:H]
            na2 = convs[l + 1][0]["W"][H:2 * H]
            h, u, v = _epi_uv(sagg, h, cp0, cp1, w2, b2,
                              na1, row(convs[l + 1][0]["b"]), na2, True)
        else:
            h = _epi_uv(sagg, h, cp0, cp1, w2, b2, None, None, None, False)

    outp = _out_mlp(h, op[0]["W"], row(op[0]["b"]), wo2, bo2)
    return outp[:N, :3]


# speedup vs baseline: 2.6700x; 1.0789x over previous
import jax
import jax.numpy as jnp
from jax import lax
from jax.experimental import pallas as pl
from jax.experimental.pallas import tpu as pltpu
from jax.experimental.pallas import tpu_sc as plsc

N = 50000
NP = 50176
E = 800000
EP = 819200
H = 64
B = 16
NBLK = 3136
EBLK = 4096

_f32 = jnp.float32


def _mesh():
    return plsc.VectorSubcoreMesh(core_axis_name="c", subcore_axis_name="s")


_SC_PARAMS = pltpu.CompilerParams(use_tc_tiling_on_sc=False)


def _count_edges(dst2):
    @pl.kernel(
        out_type=jax.ShapeDtypeStruct((2, NP, 32), _f32),
        mesh=_mesh(),
        scratch_types=[
            pltpu.VMEM((4, 2, 128), jnp.int32),
            pltpu.VMEM((128, 32), _f32),
            pltpu.VMEM((112, 32), _f32),
            pltpu.VMEM_SHARED((NP, 32), _f32),
            pltpu.SemaphoreType.DMA((4,)),
            pltpu.SemaphoreType.DMA((4,)),
        ],
        compiler_params=_SC_PARAMS,
    )
    def k(dst_hbm, cnt_hbm, idxb, ones, zb, acc, semi, semsc):
        c = lax.axis_index("c")
        s = lax.axis_index("s")
        row0 = c * 3200 + s * 200

        @pl.loop(0, 112)
        def _(i):
            for f in range(2):
                zb[i, pl.ds(f * 16, 16)] = jnp.zeros((16,), _f32)

        @pl.loop(0, 128)
        def _(i):
            for f in range(2):
                ones[i, pl.ds(f * 16, 16)] = jnp.ones((16,), _f32)

        @pl.loop(0, 28)
        def _(i):
            pltpu.sync_copy(zb, acc.at[pl.ds(s * 3136 + i * 112, 112)])

        plsc.subcore_barrier()

        def in_cp(kk, b):
            return pltpu.make_async_copy(
                dst_hbm.at[pl.ds(row0 + kk * 2, 2)], idxb.at[b], semi.at[b])

        def start_sc(b):
            for j in range(2):
                pltpu.async_copy(ones, acc.at[idxb.at[b].at[j]],
                                 semsc.at[b], add=True)

        def wait_sc(b):
            for j in range(2):
                pltpu.make_async_copy(ones, acc.at[idxb.at[b].at[j]],
                                      semsc.at[b]).wait()

        def body(kk, b, first, prefetch):
            in_cp(0, b).wait()
            start_sc(b)
            if not first:
                wait_sc((b + 1) % 4)
            if prefetch:
                in_cp(kk + 1, (b + 1) % 4).start()

        in_cp(0, 0).start()
        body(0, 0, True, True)
        body(1, 1, True, True)
        body(2, 2, True, True)

        @pl.loop(0, 24)
        def _(i):
            for q in range(4):
                body(3 + i * 4 + q, (3 + q) % 4, False, True)

        body(99, 3, False, False)
        wait_sc(1)
        wait_sc(2)
        wait_sc(3)

        plsc.subcore_barrier()
        pltpu.sync_copy(
            acc.at[pl.ds(s * 3136, 3136)],
            cnt_hbm.at[c, pl.ds(s * 3136, 3136)],
        )

    return k(dst2)


def _gather_uv(u, v, dst2, src2):
    bf16 = jnp.bfloat16

    @pl.kernel(
        out_type=jax.ShapeDtypeStruct((EP, H), bf16),
        mesh=_mesh(),
        scratch_types=[
            pltpu.VMEM((4, 2, 128), jnp.int32),
            pltpu.VMEM((4, 2, 128), jnp.int32),
            pltpu.VMEM((4, 256, H), bf16),
            pltpu.VMEM((4, 256, H), bf16),
            pltpu.SemaphoreType.DMA((4,)),
            pltpu.SemaphoreType.DMA((4,)),
            pltpu.SemaphoreType.DMA((4,)),
        ],
        compiler_params=_SC_PARAMS,
    )
    def k(u_hbm, v_hbm, dst_hbm, src_hbm, g_hbm, db, sb, ub, vb,
          semi, semg, semo):
        c = lax.axis_index("c")
        s = lax.axis_index("s")
        wid = s * 2 + c
        row0 = wid * 200
        e0 = wid * 25600

        def idx_cps(kk, b):
            r = row0 + kk * 2
            return (
                pltpu.make_async_copy(dst_hbm.at[pl.ds(r, 2)], db.at[b],
                                      semi.at[b]),
                pltpu.make_async_copy(src_hbm.at[pl.ds(r, 2)], sb.at[b],
                                      semi.at[b]),
            )

        def g_cps(b):
            cps = []
            for j in range(2):
                cps.append(pltpu.make_async_copy(
                    u_hbm.at[db.at[b].at[j]],
                    ub.at[b].at[pl.ds(j * 128, 128)], semg.at[b]))
                cps.append(pltpu.make_async_copy(
                    v_hbm.at[sb.at[b].at[j]],
                    vb.at[b].at[pl.ds(j * 128, 128)], semg.at[b]))
            return cps

        def out_cp(kk, b):
            return pltpu.make_async_copy(
                ub.at[b], g_hbm.at[pl.ds(e0 + kk * 256, 256)], semo.at[b])

        def finish(kk, b):
            for cp in g_cps(b):
                cp.wait()

            @pl.loop(0, 256, step=8)
            def _(r):
                for rr in range(8):
                    for f in range(2):
                        sl = pl.ds(f * 32, 32)
                        ub[b, r + rr, sl] = ub[b, r + rr, sl] + vb[b, r + rr, sl]

            out_cp(kk, b).start()

        def it(kk, b, wait_o, prefetch, fin_prev):
            for cp in idx_cps(0, b):
                cp.wait()
            if wait_o:
                out_cp(0, b).wait()
            for cp in g_cps(b):
                cp.start()
            if fin_prev:
                finish(kk - 3, (b + 1) % 4)
            if prefetch:
                for cp in idx_cps(kk + 1, (b + 1) % 4):
                    cp.start()

        for cp in idx_cps(0, 0):
            cp.start()
        it(0, 0, False, True, False)
        it(1, 1, False, True, False)
        it(2, 2, False, True, False)
        it(3, 3, False, True, True)

        @pl.loop(0, 23)
        def _(i):
            for q in range(4):
                it(4 + i * 4 + q, q, True, True, True)

        it(96, 0, True, True, True)
        it(97, 1, True, True, True)
        it(98, 2, True, True, True)
        it(99, 3, True, False, True)
        finish(97, 1)
        finish(98, 2)
        finish(99, 3)
        for b in range(4):
            out_cp(0, b).wait()

    return k(u, v, dst2, src2)


def _scatter_add(r, dst2):
    @pl.kernel(
        out_type=jax.ShapeDtypeStruct((NP, H), _f32),
        mesh=_mesh(),
        scratch_types=[
            pltpu.VMEM((3, 2, 128), jnp.int32),
            pltpu.VMEM((3, 256, 32), _f32),
            pltpu.VMEM((112, 32), _f32),
            pltpu.VMEM_SHARED((NP, 32), _f32),
            pltpu.SemaphoreType.DMA((3,)),
            pltpu.SemaphoreType.DMA((3,)),
        ],
        compiler_params=_SC_PARAMS,
    )
    def k(r_hbm, dst_hbm, s_hbm, idxb, rb, zb, acc, semi, semsc):
        c = lax.axis_index("c")
        s = lax.axis_index("s")

        @pl.loop(0, 112)
        def _(i):
            for f in range(2):
                zb[i, pl.ds(f * 16, 16)] = jnp.zeros((16,), _f32)

        @pl.loop(0, 28)
        def _(i):
            pltpu.sync_copy(zb, acc.at[pl.ds(s * 3136 + i * 112, 112)])

        plsc.subcore_barrier()

        def in_cps(kk, b):
            return (
                pltpu.make_async_copy(
                    dst_hbm.at[pl.ds(s * 400 + kk * 2, 2)], idxb.at[b],
                    semi.at[b]),
                pltpu.make_async_copy(
                    r_hbm.at[pl.ds(s * 51200 + kk * 256, 256),
                             pl.ds(c * 32, 32)], rb.at[b], semi.at[b]),
            )

        def start_sc(b):
            for j in range(2):
                pltpu.async_copy(rb.at[b].at[pl.ds(j * 128, 128)],
                                 acc.at[idxb.at[b].at[j]], semsc.at[b],
                                 add=True)

        def wait_sc(b):
            for j in range(2):
                pltpu.make_async_copy(rb.at[b].at[pl.ds(j * 128, 128)],
                                      acc.at[idxb.at[b].at[j]],
                                      semsc.at[b]).wait()

        def body(kk, b, first, prefetch):
            for cp in in_cps(0, b):
                cp.wait()
            start_sc(b)
            if not first:
                wait_sc((b + 1) % 3)
            if prefetch:
                for cp in in_cps(kk + 1, (b + 1) % 3):
                    cp.start()

        for cp in in_cps(0, 0):
            cp.start()
        body(0, 0, True, True)
        body(1, 1, True, True)

        @pl.loop(0, 65)
        def _(i):
            for q in range(3):
                body(2 + i * 3 + q, (2 + q) % 3, False, True)

        body(197, 2, False, True)
        body(198, 0, False, True)
        body(199, 1, False, False)
        wait_sc(0)
        wait_sc(1)

        plsc.subcore_barrier()
        pltpu.sync_copy(
            acc.at[pl.ds(s * 3136, 3136)],
            s_hbm.at[pl.ds(s * 3136, 3136), pl.ds(c * 32, 32)],
        )

    return k(r, dst2)


def _h0(xp, batchp, t2, condp, wi1, bi1, wi2, bi2, wt1, bt1, wt2, bt2,
        wc1, bc1, wc2, bc2):
    def body(x_ref, b_ref, t_ref, c_ref, wi1r, bi1r, wi2r, bi2r, wt1r, bt1r,
             wt2r, bt2r, wc1r, bc1r, wc2r, bc2r, o_ref):
        tz = jnp.maximum(t_ref[...] * wt1r[...] + bt1r[...], 0.0)
        tf = jnp.dot(tz, wt2r[...], preferred_element_type=_f32) + bt2r[...]
        cz = jnp.maximum(
            jnp.dot(c_ref[...], wc1r[...], preferred_element_type=_f32)
            + bc1r[...], 0.0)
        cf = jnp.dot(cz, wc2r[...], preferred_element_type=_f32) + bc2r[...]
        tfc = tf + cf
        z = jnp.maximum(
            jnp.dot(x_ref[...], wi1r[...], preferred_element_type=_f32)
            + bi1r[...], 0.0)
        h = jnp.dot(z, wi2r[...], preferred_element_type=_f32) + bi2r[...]
        oh = (b_ref[...] == lax.broadcasted_iota(jnp.int32, (1, B), 1)
              ).astype(_f32)
        o_ref[...] = h + jnp.dot(oh, tfc, preferred_element_type=_f32)

    full = lambda a: pl.BlockSpec(a.shape, lambda i: (0,) * a.ndim)
    return pl.pallas_call(
        body,
        grid=(NP // NBLK,),
        in_specs=[
            pl.BlockSpec((NBLK, 8), lambda i: (i, 0)),
            pl.BlockSpec((NBLK, 1), lambda i: (i, 0)),
            full(t2), full(condp),
            full(wi1), full(bi1), full(wi2), full(bi2),
            full(wt1), full(bt1), full(wt2), full(bt2),
            full(wc1), full(bc1), full(wc2), full(bc2),
        ],
        out_specs=pl.BlockSpec((NBLK, H), lambda i: (i, 0)),
        out_shape=jax.ShapeDtypeStruct((NP, H), _f32),
    )(xp, batchp, t2, condp, wi1, bi1, wi2, bi2, wt1, bt1, wt2, bt2,
      wc1, bc1, wc2, bc2)


def _edge_relu(g, eap, w1r):
    def body(g_ref, ea_ref, w_ref, o_ref):
        o_ref[...] = jnp.maximum(
            g_ref[...].astype(_f32) + ea_ref[...] * w_ref[...], 0.0)

    return pl.pallas_call(
        body,
        grid=(EP // EBLK,),
        in_specs=[
            pl.BlockSpec((EBLK, H), lambda i: (i, 0)),
            pl.BlockSpec((EBLK, 1), lambda i: (i, 0)),
            pl.BlockSpec((1, H), lambda i: (0, 0)),
        ],
        out_specs=pl.BlockSpec((EBLK, H), lambda i: (i, 0)),
        out_shape=jax.ShapeDtypeStruct((EP, H), _f32),
    )(g, eap, w1r)


def _epi_uv(sagg, h, cp0, cp1, w2, b2, a1, b1, a2, want_uv):
    def body(*refs):
        if want_uv:
            (s_ref, h_ref, c0_ref, c1_ref, w2r, b2r, a1r, b1r, a2r,
             ho, uo, vo) = refs
        else:
            s_ref, h_ref, c0_ref, c1_ref, w2r, b2r, ho = refs
        cnt = c0_ref[...].astype(_f32) + c1_ref[...].astype(_f32)
        invc = 1.0 / jnp.maximum(cnt, 1.0)
        hasb = (cnt > 0.0).astype(_f32)
        q = (jnp.dot(s_ref[...].astype(_f32) * invc, w2r[...],
                     preferred_element_type=_f32)
             + b2r[...] * hasb)
        hn = jnp.maximum(q, 0.0) + h_ref[...]
        ho[...] = hn
        if want_uv:
            uo[...] = (jnp.dot(hn, a1r[...], preferred_element_type=_f32)
                       + b1r[...]).astype(jnp.bfloat16)
            vo[...] = jnp.dot(hn, a2r[...],
                              preferred_element_type=_f32).astype(jnp.bfloat16)

    nb = pl.BlockSpec((NBLK, H), lambda i: (i, 0))
    cb = pl.BlockSpec((NBLK, 1), lambda i: (i, 0))
    full = lambda a: pl.BlockSpec(a.shape, lambda i: (0,) * a.ndim)
    if want_uv:
        in_specs = [nb, nb, cb, cb, full(w2), full(b2), full(a1), full(b1),
                    full(a2)]
        args = (sagg, h, cp0, cp1, w2, b2, a1, b1, a2)
        out_specs = [nb, nb, nb]
        out_shape = [jax.ShapeDtypeStruct((NP, H), _f32),
                     jax.ShapeDtypeStruct((NP, H), jnp.bfloat16),
                     jax.ShapeDtypeStruct((NP, H), jnp.bfloat16)]
    else:
        in_specs = [nb, nb, cb, cb, full(w2), full(b2)]
        args = (sagg, h, cp0, cp1, w2, b2)
        out_specs = nb
        out_shape = jax.ShapeDtypeStruct((NP, H), _f32)
    return pl.pallas_call(
        body, grid=(NP // NBLK,), in_specs=in_specs, out_specs=out_specs,
        out_shape=out_shape)(*args)


def _uv(h, a1, b1, a2):
    def body(h_ref, a1r, b1r, a2r, uo, vo):
        hn = h_ref[...]
        uo[...] = (jnp.dot(hn, a1r[...], preferred_element_type=_f32)
                   + b1r[...]).astype(jnp.bfloat16)
        vo[...] = jnp.dot(hn, a2r[...],
                          preferred_element_type=_f32).astype(jnp.bfloat16)

    nb = pl.BlockSpec((NBLK, H), lambda i: (i, 0))
    full = lambda a: pl.BlockSpec(a.shape, lambda i: (0,) * a.ndim)
    return pl.pallas_call(
        body,
        grid=(NP // NBLK,),
        in_specs=[nb, full(a1), full(b1), full(a2)],
        out_specs=[nb, nb],
        out_shape=[jax.ShapeDtypeStruct((NP, H), jnp.bfloat16)] * 2,
    )(h, a1, b1, a2)


def _out_mlp(h, w1, b1, w2p, b2p):
    def body(h_ref, w1r, b1r, w2r, b2r, o_ref):
        z = jnp.maximum(
            jnp.dot(h_ref[...], w1r[...], preferred_element_type=_f32)
            + b1r[...], 0.0)
        o_ref[...] = jnp.dot(z, w2r[...], preferred_element_type=_f32) + b2r[...]

    nb = pl.BlockSpec((NBLK, H), lambda i: (i, 0))
    full = lambda a: pl.BlockSpec(a.shape, lambda i: (0,) * a.ndim)
    return pl.pallas_call(
        body,
        grid=(NP // NBLK,),
        in_specs=[nb, full(w1), full(b1), full(w2p), full(b2p)],
        out_specs=pl.BlockSpec((NBLK, 8), lambda i: (i, 0)),
        out_shape=jax.ShapeDtypeStruct((NP, 8), _f32),
    )(h, w1, b1, w2p, b2p)


def kernel(x, edge_index, edge_attr, t, batch, condition, params):
    f32 = _f32
    row = lambda b: b.reshape(1, -1).astype(f32)

    src = edge_index[0]
    dst = edge_index[1]
    pad_dst = N + (jnp.arange(EP - E, dtype=jnp.int32) % (NP - N))
    dst2 = jnp.concatenate([dst, pad_dst]).reshape(-1, 128)
    src2 = jnp.concatenate([src, jnp.zeros((EP - E,), jnp.int32)]).reshape(-1, 128)
    eap = jnp.concatenate([edge_attr, jnp.zeros((EP - E, 1), f32)], axis=0)

    xp = jnp.pad(x, ((0, NP - N), (0, 8 - x.shape[1])))
    batchp = jnp.pad(batch, (0, NP - N)).reshape(-1, 1)
    t2 = t.reshape(-1, 1)
    condp = jnp.pad(condition, ((0, 0), (0, 4)))

    ip = params["input_mlp"]
    tp = params["time_mlp"]
    cp = params["cond_mlp"]
    op = params["output_mlp"]

    wi1 = jnp.pad(ip[0]["W"], ((0, 8 - ip[0]["W"].shape[0]), (0, 0)))
    wc1 = jnp.pad(cp[0]["W"], ((0, 4), (0, 0)))
    wo2 = jnp.pad(op[1]["W"], ((0, 0), (0, 8 - op[1]["W"].shape[1])))
    bo2 = jnp.pad(op[1]["b"], (0, 8 - op[1]["b"].shape[0])).reshape(1, -1)

    cnt = _count_edges(dst2)
    cp0 = cnt[0, :, 0:1]
    cp1 = cnt[1, :, 0:1]

    h = _h0(xp, batchp, t2, condp,
            wi1, row(ip[0]["b"]), ip[1]["W"], row(ip[1]["b"]),
            tp[0]["W"], row(tp[0]["b"]), tp[1]["W"], row(tp[1]["b"]),
            wc1, row(cp[0]["b"]), cp[1]["W"], row(cp[1]["b"]))

    convs = params["convs"]
    a1 = convs[0][0]["W"][:H]
    a2 = convs[0][0]["W"][H:2 * H]
    u, v = _uv(h, a1, row(convs[0][0]["b"]), a2)

    for l in range(6):
        w1r = convs[l][0]["W"][2 * H:2 * H + 1]
        g = _gather_uv(u, v, dst2, src2)
        r = _edge_relu(g, eap, w1r)
        sagg = _scatter_add(r, dst2)
        w2 = convs[l][1]["W"]
        b2 = row(convs[l][1]["b"])
        if l < 5:
            na1 = convs[l + 1][0]["W"][:H]
            na2 = convs[l + 1][0]["W"][H:2 * H]
            h, u, v = _epi_uv(sagg, h, cp0, cp1, w2, b2,
                              na1, row(convs[l + 1][0]["b"]), na2, True)
        else:
            h = _epi_uv(sagg, h, cp0, cp1, w2, b2, None, None, None, False)

    outp = _out_mlp(h, op[0]["W"], row(op[0]["b"]), wo2, bo2)
    return outp[:N, :3]
